# baseline (speedup 1.0000x reference)
import jax
import jax.numpy as jnp
from jax import lax
from jax.experimental import pallas as pl
from jax.experimental.pallas import tpu as pltpu
from jax.experimental.pallas import tpu_sc as plsc

NCAM = 8
NID = 1000
NIDP = 1024
D = 512
N = 4096
MARGIN = 0.3
WARM_UP_EPOCHS = 5

FBLK = 512
SBLK = 128
WBLK = 1024
T2_ROWS = N + 512
T2_BLK = T2_ROWS // NCAM
NSLOT = 40
NSROW = NSLOT * SBLK


def _tables_kernel(a_ref, f_ref, k_ref, camf_ref, labf_ref, an_ref, yy_ref,
                   t2_ref, anb_ref, fnb_ref, meta_ref, w_ref, rank_ref,
                   cnt_ref, acc_ref):
    i = pl.program_id(0)
    a = a_ref[0]
    ss = jnp.sum(a * a, axis=1, keepdims=True)
    an = a / (jnp.sqrt(ss) + 1e-12)
    an_ref[0, :NID, :] = an
    an_ref[0, NID:, :] = jnp.zeros((NIDP - NID, D), jnp.float32)
    anp = an_ref[0]
    ones = jnp.ones((1, D), jnp.float32)
    yy = lax.dot_general(ones, anp * anp, (((1,), (1,)), ((), ())),
                         preferred_element_type=jnp.float32)
    colid = lax.broadcasted_iota(jnp.int32, (1, NIDP), 1)
    yy_ref[0] = jnp.where(colid >= NID, jnp.float32(1e9), yy)
    anb_ref[0] = anp.astype(jnp.bfloat16)
    f = f_ref[...]
    fss = jnp.sum(f * f, axis=1, keepdims=True)
    fn = f / (jnp.sqrt(fss) + 1e-12)
    fnb_ref[...] = fn.astype(jnp.bfloat16)
    xx = jnp.sum(fn * fn, axis=1, keepdims=True)
    t2_ref[:FBLK, :] = 0.5 * fn
    t2_ref[FBLK:, :] = jnp.zeros((T2_BLK - FBLK, D), jnp.float32)
    keys = k_ref[...]
    kv = (i * WBLK + lax.broadcasted_iota(jnp.int32, (1, WBLK), 1)
          ).astype(jnp.float32)
    eq = keys == kv
    nidx = lax.broadcasted_iota(jnp.int32, (N, WBLK), 0).astype(jnp.float32)
    cand = jnp.where(eq, nidx, jnp.float32(-1.0))
    w_ref[0] = jnp.max(cand, axis=0, keepdims=True)
    @pl.when(i == 0)
    def _():
        acc_ref[...] = jnp.zeros((1, NCAM), jnp.float32)

    camq = camf_ref[...]
    lab = labf_ref[...]
    c16 = lax.broadcasted_iota(jnp.int32, (FBLK, 128), 1)
    meta_ref[...] = jnp.where(c16 == 0, lab,
                              jnp.where(c16 == 1, xx, jnp.float32(0.0)))
    cid = lax.broadcasted_iota(jnp.int32, (FBLK, NCAM), 1).astype(jnp.float32)
    oneh = jnp.where(camq == cid, jnp.float32(1.0), jnp.float32(0.0))
    ri = lax.broadcasted_iota(jnp.int32, (FBLK, FBLK), 0)
    ci = lax.broadcasted_iota(jnp.int32, (FBLK, FBLK), 1)
    tril = jnp.where(ri > ci, jnp.float32(1.0), jnp.float32(0.0))
    pre = lax.dot_general(tril, oneh, (((1,), (0,)), ((), ())),
                          preferred_element_type=jnp.float32)
    rk = pre + acc_ref[...]
    rank_ref[...] = jnp.sum(jnp.where(oneh > 0, rk, jnp.float32(0.0)),
                            axis=1, keepdims=True)
    acc_ref[...] += jnp.sum(oneh, axis=0, keepdims=True)
    cnt_ref[...] = acc_ref[...]


def _loss_kernel(cam_t, len_t, anb_ref, yy_ref, f_ref, meta_ref, loss_ref):
    t = pl.program_id(0)

    @pl.when(t == 0)
    def _():
        loss_ref[0, 0] = jnp.float32(0.0)

    f = f_ref[...]
    a = anb_ref[0]
    s = lax.dot_general(f, a, (((1,), (1,)), ((), ())),
                        preferred_element_type=jnp.float32)
    tt = yy_ref[0] - 2.0 * s
    meta = meta_ref[...]
    lab = meta[:, 0:1]
    xx = meta[:, 1:2]
    col = lax.broadcasted_iota(jnp.int32, (SBLK, NIDP), 1).astype(jnp.float32)
    oneh = col == lab
    mn = jnp.min(jnp.where(oneh, jnp.float32(1e30), tt), axis=1,
                 keepdims=True)
    tl = jnp.sum(jnp.where(oneh, tt, jnp.float32(0.0)), axis=1,
                 keepdims=True)
    dn = jnp.maximum(xx + mn, jnp.float32(1e-12))
    dp = jnp.maximum(xx + tl, jnp.float32(1e-12))
    hinge = jnp.maximum(dp - dn + jnp.float32(MARGIN), jnp.float32(0.0))
    rowi = lax.broadcasted_iota(jnp.int32, (SBLK, 1), 0)
    contrib = jnp.where(rowi < len_t[t], hinge, jnp.float32(0.0))
    loss_ref[0, 0] += jnp.sum(contrib)


def _sc_kernel(rank_hbm, cam_hbm, aoff_hbm, fnbp_hbm, meta_hbm, win_hbm,
               t2_hbm, fs_hbm, ms_hbm, outb_hbm, rank_v, camv_v, aoff_v,
               sidx_v, bufp, bufm, win_v, idx_v, buf, sem):
    c = lax.axis_index("c")
    s = lax.axis_index("s")
    wid = s * 2 + c
    base = wid * 128
    pltpu.sync_copy(rank_hbm.at[pl.ds(base, 128)], rank_v)
    pltpu.sync_copy(cam_hbm.at[pl.ds(base, 128)], camv_v)
    pltpu.sync_copy(aoff_hbm, aoff_v)
    av = aoff_v[...]
    for v in range(8):
        cam16 = camv_v[pl.ds(v * 16, 16)]
        rk16 = rank_v[pl.ds(v * 16, 16)].astype(jnp.int32)
        ao16 = lax.gather(
            av, cam16.reshape(16, 1),
            lax.GatherDimensionNumbers(offset_dims=(),
                                       collapsed_slice_dims=(0,),
                                       start_index_map=(0,)),
            slice_sizes=(1,),
            mode=lax.GatherScatterMode.PROMISE_IN_BOUNDS)
        sidx_v[pl.ds(v * 16, 16)] = ao16 + rk16
    pltpu.sync_copy(fnbp_hbm.at[pl.ds(base, 128)], bufp)
    pltpu.async_copy(bufp, fs_hbm.at[sidx_v], sem).wait()
    pltpu.sync_copy(meta_hbm.at[pl.ds(base, 128)], bufm)
    pltpu.async_copy(bufm, ms_hbm.at[sidx_v], sem).wait()
    cam = wid // 4
    lb = (wid % 4) * 256
    for ch in range(2):
        l0 = lb + ch * 128
        k0 = cam * NIDP + l0
        pltpu.sync_copy(win_hbm.at[pl.ds(k0, 128)], win_v)
        for v in range(8):
            w = win_v[pl.ds(v * 16, 16)]
            kv = k0 + v * 16 + lax.iota(jnp.int32, 16)
            has = w >= 0.0
            wi = w.astype(jnp.int32)
            wi = wi + (wi >> 9) * (T2_BLK - FBLK)
            jz = FBLK + T2_BLK * (kv & 7) + ((kv >> 3) & 63)
            idx_v[pl.ds(v * 16, 16)] = jnp.where(has, wi, jz)
        pltpu.async_copy(t2_hbm.at[idx_v], buf, sem).wait()
        r0 = cam * NID + l0
        last = jnp.logical_and(wid % 4 == 3, ch == 1)

        @pl.when(jnp.logical_not(last))
        def _():
            pltpu.sync_copy(buf, outb_hbm.at[pl.ds(r0, 128)])

        @pl.when(last)
        def _():
            pltpu.sync_copy(buf.at[pl.ds(0, NID - 896)],
                            outb_hbm.at[pl.ds(r0, NID - 896)])


def _combine_kernel(an_ref, win_ref, b_ref, out_ref):
    a = an_ref[0]
    w = win_ref[0]
    scale = jnp.where(w >= 0.0, jnp.float32(0.5), jnp.float32(1.0))
    rowa = a * scale
    out_ref[0] = rowa[:NID, :] + b_ref[...]


def _build_tables(intra_anchors, features, keys_f, camf, labf):
    return pl.pallas_call(
        _tables_kernel,
        grid=(NCAM,),
        in_specs=[
            pl.BlockSpec((1, NID, D), lambda i: (i, 0, 0)),
            pl.BlockSpec((FBLK, D), lambda i: (i, 0)),
            pl.BlockSpec((N, 1), lambda i: (0, 0)),
            pl.BlockSpec((FBLK, 1), lambda i: (i, 0)),
            pl.BlockSpec((FBLK, 1), lambda i: (i, 0)),
        ],
        out_specs=[
            pl.BlockSpec((1, NIDP, D), lambda i: (i, 0, 0)),
            pl.BlockSpec((1, 1, NIDP), lambda i: (i, 0, 0)),
            pl.BlockSpec((T2_BLK, D), lambda i: (i, 0)),
            pl.BlockSpec((1, NIDP, D), lambda i: (i, 0, 0)),
            pl.BlockSpec((FBLK, D), lambda i: (i, 0)),
            pl.BlockSpec((FBLK, 128), lambda i: (i, 0)),
            pl.BlockSpec((1, 1, WBLK), lambda i: (i, 0, 0)),
            pl.BlockSpec((FBLK, 1), lambda i: (i, 0)),
            pl.BlockSpec((1, NCAM), lambda i: (0, 0)),
        ],
        out_shape=[
            jax.ShapeDtypeStruct((NCAM, NIDP, D), jnp.float32),
            jax.ShapeDtypeStruct((NCAM, 1, NIDP), jnp.float32),
            jax.ShapeDtypeStruct((T2_ROWS, D), jnp.float32),
            jax.ShapeDtypeStruct((NCAM, NIDP, D), jnp.bfloat16),
            jax.ShapeDtypeStruct((N, D), jnp.bfloat16),
            jax.ShapeDtypeStruct((N, 128), jnp.float32),
            jax.ShapeDtypeStruct((NCAM, 1, WBLK), jnp.float32),
            jax.ShapeDtypeStruct((N, 1), jnp.float32),
            jax.ShapeDtypeStruct((1, NCAM), jnp.float32),
        ],
        scratch_shapes=[pltpu.VMEM((1, NCAM), jnp.float32)],
    )(intra_anchors, features, keys_f, camf, labf)


def _loss_sum(cam_t, len_t, anb, yy, fsorted, msorted):
    grid_spec = pltpu.PrefetchScalarGridSpec(
        num_scalar_prefetch=2,
        grid=(NSLOT,),
        in_specs=[
            pl.BlockSpec((1, NIDP, D), lambda t, ct, lt: (ct[t], 0, 0)),
            pl.BlockSpec((1, 1, NIDP), lambda t, ct, lt: (ct[t], 0, 0)),
            pl.BlockSpec((SBLK, D), lambda t, ct, lt: (t, 0)),
            pl.BlockSpec((SBLK, 128), lambda t, ct, lt: (t, 0)),
        ],
        out_specs=pl.BlockSpec((1, 1), lambda t, ct, lt: (0, 0),
                               memory_space=pltpu.SMEM),
    )
    return pl.pallas_call(
        _loss_kernel,
        grid_spec=grid_spec,
        out_shape=jax.ShapeDtypeStruct((1, 1), jnp.float32),
    )(cam_t, len_t, anb, yy, fsorted, msorted)


def _sc_run(rank_flat, cams0, aoff16, fnbp, meta, winner_flat, t2):
    mesh = plsc.VectorSubcoreMesh(core_axis_name="c", subcore_axis_name="s")
    run = pl.kernel(
        _sc_kernel,
        out_type=(
            jax.ShapeDtypeStruct((NSROW, D // 2), jnp.int32),
            jax.ShapeDtypeStruct((NSROW, 128), jnp.float32),
            jax.ShapeDtypeStruct((NCAM * NID, D), jnp.float32),
        ),
        mesh=mesh,
        scratch_types=[
            pltpu.VMEM((128,), jnp.float32),
            pltpu.VMEM((128,), jnp.int32),
            pltpu.VMEM((16,), jnp.int32),
            pltpu.VMEM((128,), jnp.int32),
            pltpu.VMEM((128, D // 2), jnp.int32),
            pltpu.VMEM((128, 128), jnp.float32),
            pltpu.VMEM((128,), jnp.float32),
            pltpu.VMEM((128,), jnp.int32),
            pltpu.VMEM((128, D), jnp.float32),
            pltpu.SemaphoreType.DMA,
        ],
    )
    return run(rank_flat, cams0, aoff16, fnbp, meta, winner_flat, t2)


def _combine(ann, winner_col, outb):
    return pl.pallas_call(
        _combine_kernel,
        grid=(NCAM,),
        in_specs=[
            pl.BlockSpec((1, NIDP, D), lambda i: (i, 0, 0)),
            pl.BlockSpec((1, NIDP, 1), lambda i: (i, 0, 0)),
            pl.BlockSpec((NID, D), lambda i: (i, 0)),
        ],
        out_specs=pl.BlockSpec((1, NID, D), lambda i: (i, 0, 0)),
        out_shape=jax.ShapeDtypeStruct((NCAM, NID, D), jnp.float32),
    )(ann, winner_col, outb)


def kernel(features, labels, cams, intra_anchors, cross_anchors, epoch):
    labels0 = (labels - 1).astype(jnp.int32)
    cams0 = (cams - 1).astype(jnp.int32)
    keys_f = (cams0 * NIDP + labels0).astype(jnp.float32).reshape(N, 1)
    camf = cams0.astype(jnp.float32).reshape(N, 1)
    labf = labels0.astype(jnp.float32).reshape(N, 1)

    ann, yy, t2, anb, fnb, meta, winner, rank, cnt = _build_tables(
        intra_anchors, features, keys_f, camf, labf)

    def warm_fn(_):
        cnt_i = cnt.reshape(NCAM).astype(jnp.int32)
        nb = (cnt_i + (SBLK - 1)) // SBLK
        cum = jnp.cumsum(nb)
        cstart = cum - nb
        aoff = (SBLK * cstart).astype(jnp.int32)
        tarr = jnp.arange(NSLOT, dtype=jnp.int32)
        camc = jnp.clip(jnp.searchsorted(cum, tarr, side="right"),
                        0, NCAM - 1).astype(jnp.int32)
        b_in = tarr - cstart[camc]
        len_raw = jnp.clip(cnt_i[camc] - SBLK * b_in, 0, SBLK)
        valid = tarr < cum[-1]
        cam_t = jnp.where(valid, camc, 0).astype(jnp.int32)
        len_t = jnp.where(valid, len_raw, 0).astype(jnp.int32)
        fnbp = lax.bitcast_convert_type(fnb.reshape(N, D // 2, 2), jnp.int32)
        aoff16 = jnp.pad(aoff, (0, 16 - NCAM))
        fs_i32, ms, outb = _sc_run(rank.reshape(N), cams0, aoff16, fnbp,
                                   meta, winner.reshape(NCAM * NIDP), t2)
        fsorted = lax.bitcast_convert_type(
            fs_i32, jnp.bfloat16).reshape(NSROW, D)
        loss_sum = _loss_sum(cam_t, len_t, anb, yy, fsorted, ms)
        new_anchors = _combine(ann, winner.reshape(NCAM, NIDP, 1), outb)
        loss = loss_sum[0, 0] * jnp.float32(1.0 / N)
        return loss, new_anchors

    def cold_fn(_):
        return jnp.float32(0.0), ann[:, :NID, :]

    warm = epoch <= WARM_UP_EPOCHS
    loss, new_anchors = lax.cond(warm, warm_fn, cold_fn, 0)
    return (loss, new_anchors, cross_anchors)


# speedup vs baseline: 1.7141x; 1.7141x over previous
import jax
import jax.numpy as jnp
from jax import lax
from jax.experimental import pallas as pl
from jax.experimental.pallas import tpu as pltpu
from jax.experimental.pallas import tpu_sc as plsc

NCAM = 8
NID = 1000
NIDP = 1024
D = 512
N = 4096
MARGIN = 0.3
WARM_UP_EPOCHS = 5

FBLK = 512
SBLK = 1024
WBLK = 1024
T2_ROWS = N + 512
T2_BLK = T2_ROWS // NCAM


def _norm_tables_kernel(a_ref, f_ref, k_ref, an_ref, yy_ref, t2_ref,
                        anb_ref, fnb_ref, xx_ref, w_ref):
    a = a_ref[0]
    ss = jnp.sum(a * a, axis=1, keepdims=True)
    an = a / (jnp.sqrt(ss) + 1e-12)
    an_ref[0, :NID, :] = an
    an_ref[0, NID:, :] = jnp.zeros((NIDP - NID, D), jnp.float32)
    anp = an_ref[0]
    ones = jnp.ones((1, D), jnp.float32)
    yy = lax.dot_general(ones, anp * anp, (((1,), (1,)), ((), ())),
                         preferred_element_type=jnp.float32)
    colid = lax.broadcasted_iota(jnp.int32, (1, NIDP), 1)
    yy_ref[0] = jnp.where(colid >= NID, jnp.float32(1e9), yy)
    anb_ref[0] = anp.astype(jnp.bfloat16)
    f = f_ref[...]
    fss = jnp.sum(f * f, axis=1, keepdims=True)
    fn = f / (jnp.sqrt(fss) + 1e-12)
    fnb_ref[...] = fn.astype(jnp.bfloat16)
    xx_ref[...] = jnp.sum(fn * fn, axis=1, keepdims=True)
    t2_ref[:FBLK, :] = 0.5 * fn
    t2_ref[FBLK:, :] = jnp.zeros((T2_BLK - FBLK, D), jnp.float32)
    b = pl.program_id(0)
    kv = (b * WBLK + lax.broadcasted_iota(jnp.int32, (1, WBLK), 1)
          ).astype(jnp.float32)
    keys = k_ref[...]
    eq = keys == kv
    nidx = lax.broadcasted_iota(jnp.int32, (N, WBLK), 0).astype(jnp.float32)
    cand = jnp.where(eq, nidx, jnp.float32(-1.0))
    w_ref[0] = jnp.max(cand, axis=0, keepdims=True)


def _loss_kernel(an_ref, yy_ref, f_ref, xx_ref, lab_ref, cam_ref, loss_ref):
    i = pl.program_id(0)
    j = pl.program_id(1)

    @pl.when(jnp.logical_and(i == 0, j == 0))
    def _():
        loss_ref[0, 0] = jnp.float32(0.0)

    f = f_ref[...]
    a = an_ref[0]
    s = lax.dot_general(f, a, (((1,), (1,)), ((), ())),
                        preferred_element_type=jnp.float32)
    xx = xx_ref[...]
    tt = yy_ref[0] - 2.0 * s
    lab = lab_ref[...]
    col = lax.broadcasted_iota(jnp.int32, (SBLK, NIDP), 1).astype(jnp.float32)
    oneh = col == lab
    mn = jnp.min(jnp.where(oneh, jnp.float32(1e30), tt), axis=1,
                 keepdims=True)
    tl = jnp.sum(jnp.where(oneh, tt, jnp.float32(0.0)), axis=1,
                 keepdims=True)
    dn = jnp.maximum(xx + mn, jnp.float32(1e-12))
    dp = jnp.maximum(xx + tl, jnp.float32(1e-12))
    cam = cam_ref[...]
    hinge = jnp.maximum(dp - dn + jnp.float32(MARGIN), jnp.float32(0.0))
    contrib = jnp.where(cam == i.astype(jnp.float32), hinge,
                        jnp.float32(0.0))
    loss_ref[0, 0] += jnp.sum(contrib)


def _update_kernel(win_hbm, t2_hbm, outb_hbm, win_v, idx_v, buf, sem):
    c = lax.axis_index("c")
    s = lax.axis_index("s")
    wid = s * 2 + c
    cam = wid // 4
    lb = (wid % 4) * 256
    for ch in range(2):
        l0 = lb + ch * 128
        k0 = cam * NIDP + l0
        pltpu.sync_copy(win_hbm.at[pl.ds(k0, 128)], win_v)
        for v in range(8):
            w = win_v[pl.ds(v * 16, 16)]
            kv = k0 + v * 16 + lax.iota(jnp.int32, 16)
            has = w >= 0.0
            wi = w.astype(jnp.int32)
            wi = wi + (wi >> 9) * (T2_BLK - FBLK)
            jz = FBLK + T2_BLK * (kv & 7) + ((kv >> 3) & 63)
            idx_v[pl.ds(v * 16, 16)] = jnp.where(has, wi, jz)
        pltpu.async_copy(t2_hbm.at[idx_v], buf, sem).wait()
        r0 = cam * NID + l0
        last = jnp.logical_and(wid % 4 == 3, ch == 1)

        @pl.when(jnp.logical_not(last))
        def _():
            pltpu.sync_copy(buf, outb_hbm.at[pl.ds(r0, 128)])

        @pl.when(last)
        def _():
            pltpu.sync_copy(buf.at[pl.ds(0, NID - 896)],
                            outb_hbm.at[pl.ds(r0, NID - 896)])


def _combine_kernel(an_ref, win_ref, b_ref, out_ref):
    a = an_ref[0]
    w = win_ref[0]
    scale = jnp.where(w >= 0.0, jnp.float32(0.5), jnp.float32(1.0))
    rowa = a * scale
    out_ref[0] = rowa[:NID, :] + b_ref[...]


def _build_tables(intra_anchors, features, keys_f):
    return pl.pallas_call(
        _norm_tables_kernel,
        grid=(NCAM,),
        in_specs=[
            pl.BlockSpec((1, NID, D), lambda i: (i, 0, 0)),
            pl.BlockSpec((FBLK, D), lambda i: (i, 0)),
            pl.BlockSpec((N, 1), lambda i: (0, 0)),
        ],
        out_specs=[
            pl.BlockSpec((1, NIDP, D), lambda i: (i, 0, 0)),
            pl.BlockSpec((1, 1, NIDP), lambda i: (i, 0, 0)),
            pl.BlockSpec((T2_BLK, D), lambda i: (i, 0)),
            pl.BlockSpec((1, NIDP, D), lambda i: (i, 0, 0)),
            pl.BlockSpec((FBLK, D), lambda i: (i, 0)),
            pl.BlockSpec((FBLK, 1), lambda i: (i, 0)),
            pl.BlockSpec((1, 1, WBLK), lambda i: (i, 0, 0)),
        ],
        out_shape=[
            jax.ShapeDtypeStruct((NCAM, NIDP, D), jnp.float32),
            jax.ShapeDtypeStruct((NCAM, 1, NIDP), jnp.float32),
            jax.ShapeDtypeStruct((T2_ROWS, D), jnp.float32),
            jax.ShapeDtypeStruct((NCAM, NIDP, D), jnp.bfloat16),
            jax.ShapeDtypeStruct((N, D), jnp.bfloat16),
            jax.ShapeDtypeStruct((N, 1), jnp.float32),
            jax.ShapeDtypeStruct((NCAM, 1, WBLK), jnp.float32),
        ],
    )(intra_anchors, features, keys_f)


def _loss_sum(anb, yy, fnb, xx, labf, camf):
    return pl.pallas_call(
        _loss_kernel,
        grid=(NCAM, N // SBLK),
        in_specs=[
            pl.BlockSpec((1, NIDP, D), lambda i, j: (i, 0, 0)),
            pl.BlockSpec((1, 1, NIDP), lambda i, j: (i, 0, 0)),
            pl.BlockSpec((SBLK, D), lambda i, j: (j, 0)),
            pl.BlockSpec((SBLK, 1), lambda i, j: (j, 0)),
            pl.BlockSpec((SBLK, 1), lambda i, j: (j, 0)),
            pl.BlockSpec((SBLK, 1), lambda i, j: (j, 0)),
        ],
        out_specs=pl.BlockSpec((1, 1), lambda i, j: (0, 0),
                               memory_space=pltpu.SMEM),
        out_shape=jax.ShapeDtypeStruct((1, 1), jnp.float32),
    )(anb, yy, fnb, xx, labf, camf)


def _update_rows(winner_flat, t2):
    mesh = plsc.VectorSubcoreMesh(core_axis_name="c", subcore_axis_name="s")
    run = pl.kernel(
        _update_kernel,
        out_type=jax.ShapeDtypeStruct((NCAM * NID, D), jnp.float32),
        mesh=mesh,
        scratch_types=[
            pltpu.VMEM((128,), jnp.float32),
            pltpu.VMEM((128,), jnp.int32),
            pltpu.VMEM((128, D), jnp.float32),
            pltpu.SemaphoreType.DMA,
        ],
    )
    return run(winner_flat, t2)


def _combine(ann, winner_col, outb):
    return pl.pallas_call(
        _combine_kernel,
        grid=(NCAM,),
        in_specs=[
            pl.BlockSpec((1, NIDP, D), lambda i: (i, 0, 0)),
            pl.BlockSpec((1, NIDP, 1), lambda i: (i, 0, 0)),
            pl.BlockSpec((NID, D), lambda i: (i, 0)),
        ],
        out_specs=pl.BlockSpec((1, NID, D), lambda i: (i, 0, 0)),
        out_shape=jax.ShapeDtypeStruct((NCAM, NID, D), jnp.float32),
    )(ann, winner_col, outb)


def kernel(features, labels, cams, intra_anchors, cross_anchors, epoch):
    labels0 = (labels - 1).astype(jnp.int32)
    cams0 = (cams - 1).astype(jnp.int32)
    keys_f = (cams0 * NIDP + labels0).astype(jnp.float32).reshape(N, 1)
    labf = labels0.astype(jnp.float32).reshape(N, 1)
    camf = cams0.astype(jnp.float32).reshape(N, 1)

    ann, yy, t2, anb, fnb, xx, winner = _build_tables(
        intra_anchors, features, keys_f)

    def warm_fn(_):
        loss_sum = _loss_sum(anb, yy, fnb, xx, labf, camf)
        outb = _update_rows(winner.reshape(NCAM * NIDP), t2)
        new_anchors = _combine(ann, winner.reshape(NCAM, NIDP, 1), outb)
        loss = loss_sum[0, 0] * jnp.float32(1.0 / N)
        return loss, new_anchors

    def cold_fn(_):
        return jnp.float32(0.0), ann[:, :NID, :]

    warm = epoch <= WARM_UP_EPOCHS
    loss, new_anchors = lax.cond(warm, warm_fn, cold_fn, 0)
    return (loss, new_anchors, cross_anchors)
